# baseline (speedup 1.0000x reference)
import jax
import jax.numpy as jnp
from jax import lax
from jax.experimental import pallas as pl
from jax.experimental.pallas import tpu as pltpu
from jax.experimental.pallas import tpu_sc as plsc

S = 50
V = 39
D = 32
B = 16384
NW = 32
BW = B // NW
SV = S * V
DT = D // 8
BT = BW // 128
STAGE = DT * BT * 8 * 128


def _table_body(pe_ref, ew_ref, out_ref):
    pe_t = pe_ref[...].T
    ew_t = ew_ref[...].T
    out_ref[...] = pe_t[:, :, None] + ew_t[:, None, :]


def _fused_table_t(pe, ew):
    t = pl.pallas_call(
        _table_body,
        out_shape=jax.ShapeDtypeStruct((D, S, V), jnp.float32),
    )(pe, ew)
    return t.reshape(D * S * V)


def _sc_body(x_hbm, tt_hbm, out_hbm, xv, ttv, stage, osem0, osem1):
    wid = lax.axis_index("s") * 2 + lax.axis_index("c")
    b0 = wid * BW
    pltpu.sync_copy(tt_hbm, ttv)
    pltpu.sync_copy(x_hbm.at[pl.ds(b0 * S, BW * S)], xv)
    i50 = lax.iota(jnp.int32, 16) * S

    def fill(s, slot):
        @plsc.parallel_loop(0, BW // 16, unroll=4)
        def _(jj):
            tb_off = (jj // 8) * 1024 + (jj % 8) * 16
            xg = plsc.load_gather(xv, [i50 + (jj * (16 * S) + s)])
            f = xg + s * V
            for d in range(D):
                tv = plsc.load_gather(ttv, [f + d * SV])
                off = tb_off + (d // 8) * (BT * 1024) + (d % 8) * 128
                stage[slot, pl.ds(off, 16)] = tv

    def flush(s, slot, sem):
        for dt in range(DT):
            pltpu.async_copy(
                stage.at[slot, pl.ds(dt * (BT * 1024), BT * 1024)],
                out_hbm.at[pl.ds(((s * DT + dt) * 128 + wid * BT) * 1024,
                                 BT * 1024)],
                sem,
            )

    def drain(slot, sem):
        pltpu.make_async_copy(
            out_hbm.at[pl.ds(0, STAGE)], stage.at[slot], sem
        ).wait()

    def do_pair(p, carry):
        @pl.when(p > 0)
        def _():
            drain(0, osem0)

        fill(2 * p, 0)
        flush(2 * p, 0, osem0)

        @pl.when(p > 0)
        def _():
            drain(1, osem1)

        fill(2 * p + 1, 1)
        flush(2 * p + 1, 1, osem1)
        return carry

    lax.fori_loop(0, S // 2, do_pair, 0)
    drain(0, osem0)
    drain(1, osem1)


def _sc_lookup(x_flat, ttf):
    mesh = plsc.VectorSubcoreMesh(core_axis_name="c", subcore_axis_name="s")
    fn = pl.kernel(
        _sc_body,
        mesh=mesh,
        out_type=jax.ShapeDtypeStruct((B * S * D,), jnp.float32),
        scratch_types=[
            pltpu.VMEM((BW * S,), jnp.int32),
            pltpu.VMEM((D * S * V,), jnp.float32),
            pltpu.VMEM((2, STAGE), jnp.float32),
            pltpu.SemaphoreType.DMA,
            pltpu.SemaphoreType.DMA,
        ],
        compiler_params=pltpu.CompilerParams(
            use_tc_tiling_on_sc=False, needs_layout_passes=False
        ),
    )
    return fn(x_flat, ttf)


def kernel(x, embed_weight, pe):
    ttf = _fused_table_t(pe.astype(jnp.float32), embed_weight.astype(jnp.float32))
    x_flat = x.reshape(-1).astype(jnp.int32)
    flat = _sc_lookup(x_flat, ttf)
    o5 = flat.reshape(S, DT, B // 128, 8, 128)
    return o5.transpose(2, 4, 0, 1, 3).reshape(B, S, D)


# speedup vs baseline: 1.3385x; 1.3385x over previous
import jax
import jax.numpy as jnp
from jax import lax
from jax.experimental import pallas as pl
from jax.experimental.pallas import tpu as pltpu
from jax.experimental.pallas import tpu_sc as plsc

S = 50
V = 39
D = 32
B = 16384
NW = 32
BW = B // NW
SV = S * V
DT = D // 8
BT = BW // 128
STAGE = DT * BT * 8 * 128


def _table_body(pe_ref, ew_ref, out_ref):
    pe_t = pe_ref[...].T
    ew_t = ew_ref[...].T
    out_ref[...] = pe_t[:, :, None] + ew_t[:, None, :]


def _fused_table_t(pe, ew):
    t = pl.pallas_call(
        _table_body,
        out_shape=jax.ShapeDtypeStruct((D, S, V), jnp.float32),
    )(pe, ew)
    return t.reshape(D * S * V)


def _sc_body(x_hbm, tt_hbm, out_hbm, xv, ttv, stage, osem0, osem1):
    wid = lax.axis_index("s") * 2 + lax.axis_index("c")
    b0 = wid * BW
    pltpu.sync_copy(tt_hbm, ttv)
    pltpu.sync_copy(x_hbm.at[pl.ds(b0 * S, BW * S)], xv)
    i50 = lax.iota(jnp.int32, 16) * S

    def fill(s, slot):
        @plsc.parallel_loop(0, BW // 16, unroll=2)
        def _(jj):
            tb_off = (jj // 8) * 1024 + (jj % 8) * 16
            xg = plsc.load_gather(xv, [i50 + (jj * (16 * S) + s)])
            idx = xg + s * V
            for d in range(D):
                tv = plsc.load_gather(ttv, [idx])
                off = tb_off + (d // 8) * (BT * 1024) + (d % 8) * 128
                stage[slot, pl.ds(off, 16)] = tv
                idx = idx + SV

    def flush(s, slot, sem):
        for dt in range(DT):
            pltpu.async_copy(
                stage.at[slot, pl.ds(dt * (BT * 1024), BT * 1024)],
                out_hbm.at[pl.ds(((s * DT + dt) * 128 + wid * BT) * 1024,
                                 BT * 1024)],
                sem,
            )

    def drain(slot, sem):
        pltpu.make_async_copy(
            out_hbm.at[pl.ds(0, STAGE)], stage.at[slot], sem
        ).wait()

    def do_pair(p, carry):
        @pl.when(p > 0)
        def _():
            drain(0, osem0)

        fill(2 * p, 0)
        flush(2 * p, 0, osem0)

        @pl.when(p > 0)
        def _():
            drain(1, osem1)

        fill(2 * p + 1, 1)
        flush(2 * p + 1, 1, osem1)
        return carry

    lax.fori_loop(0, S // 2, do_pair, 0)
    drain(0, osem0)
    drain(1, osem1)


def _sc_lookup(x_flat, ttf):
    mesh = plsc.VectorSubcoreMesh(core_axis_name="c", subcore_axis_name="s")
    fn = pl.kernel(
        _sc_body,
        mesh=mesh,
        out_type=jax.ShapeDtypeStruct((B * S * D,), jnp.float32),
        scratch_types=[
            pltpu.VMEM((BW * S,), jnp.int32),
            pltpu.VMEM((D * S * V,), jnp.float32),
            pltpu.VMEM((2, STAGE), jnp.float32),
            pltpu.SemaphoreType.DMA,
            pltpu.SemaphoreType.DMA,
        ],
        compiler_params=pltpu.CompilerParams(
            use_tc_tiling_on_sc=False, needs_layout_passes=False
        ),
    )
    return fn(x_flat, ttf)


def kernel(x, embed_weight, pe):
    ttf = _fused_table_t(pe.astype(jnp.float32), embed_weight.astype(jnp.float32))
    x_flat = x.reshape(-1).astype(jnp.int32)
    flat = _sc_lookup(x_flat, ttf)
    o5 = flat.reshape(S, DT, B // 128, 8, 128)
    return o5.transpose(2, 4, 0, 1, 3).reshape(B, S, D)
